# baseline (speedup 1.0000x reference)
import functools

import jax
import jax.numpy as jnp
from jax import lax
from jax.experimental import pallas as pl
from jax.experimental.pallas import tpu as pltpu
from jax.experimental.pallas import tpu_sc as plsc

N = 16777216
NBINS = 10
NCORES = 2
NSUB = 16
NWORK = NCORES * NSUB
PER_W = N // NWORK
CHUNK = 8192
NBLK = PER_W // CHUNK
NVEC = CHUNK // 16

_mesh = plsc.VectorSubcoreMesh(core_axis_name="c", subcore_axis_name="s")


@functools.partial(
    pl.kernel,
    mesh=_mesh,
    out_type=jax.ShapeDtypeStruct((3 * NWORK, 16), jnp.float32),
    compiler_params=pltpu.CompilerParams(needs_layout_passes=False),
    scratch_types=[
        pltpu.VMEM((2, CHUNK), jnp.float32),
        pltpu.VMEM((2, CHUNK), jnp.int32),
        pltpu.VMEM((2, CHUNK), jnp.int32),
        pltpu.VMEM((NBINS, 16), jnp.float32),
        pltpu.VMEM((NBINS, 16), jnp.float32),
        pltpu.VMEM((NBINS, 16), jnp.float32),
        pltpu.VMEM((16,), jnp.float32),
        pltpu.SemaphoreType.DMA,
        pltpu.SemaphoreType.DMA,
        pltpu.SemaphoreType.DMA,
        pltpu.SemaphoreType.DMA,
        pltpu.SemaphoreType.DMA,
        pltpu.SemaphoreType.DMA,
    ],
)
def _ece_partials(c_hbm, p_hbm, l_hbm, out_hbm, cbuf, pbuf, lbuf,
                  hc, ha, hs, row, sc0, sc1, sp0, sp1, sl0, sl1):
    wid = lax.axis_index("s") * NCORES + lax.axis_index("c")
    base = wid * PER_W
    zero = jnp.zeros((16,), jnp.float32)
    ones = jnp.ones((16,), jnp.float32)
    lane = lax.iota(jnp.int32, 16)
    for b in range(NBINS):
        hc[b] = zero
        ha[b] = zero
        hs[b] = zero

    csem = (sc0, sc1)
    psem = (sp0, sp1)
    lsem = (sl0, sl1)

    def _start(g, slot):
        off = base + g * CHUNK
        pltpu.async_copy(c_hbm.at[pl.ds(off, CHUNK)], cbuf.at[slot], csem[slot])
        pltpu.async_copy(p_hbm.at[pl.ds(off, CHUNK)], pbuf.at[slot], psem[slot])
        pltpu.async_copy(l_hbm.at[pl.ds(off, CHUNK)], lbuf.at[slot], lsem[slot])

    def _wait(slot):
        pltpu.make_async_copy(c_hbm.at[pl.ds(0, CHUNK)], cbuf.at[slot], csem[slot]).wait()
        pltpu.make_async_copy(p_hbm.at[pl.ds(0, CHUNK)], pbuf.at[slot], psem[slot]).wait()
        pltpu.make_async_copy(l_hbm.at[pl.ds(0, CHUNK)], lbuf.at[slot], lsem[slot]).wait()

    _start(0, 0)
    _start(1, 1)

    def _compute(slot):
        def vbody(v, carry):
            off = v * 16
            c = cbuf[slot, pl.ds(off, 16)]
            p = pbuf[slot, pl.ds(off, 16)]
            l = lbuf[slot, pl.ds(off, 16)]
            acc = jnp.where(p == l, 1.0, 0.0).astype(jnp.float32)
            bi = jnp.minimum((c * 10.0).astype(jnp.int32), NBINS - 1)
            plsc.addupdate_scatter(hc, [bi, lane], ones)
            plsc.addupdate_scatter(ha, [bi, lane], acc)
            plsc.addupdate_scatter(hs, [bi, lane], c)
            return carry
        lax.fori_loop(0, NVEC, vbody, 0)

    def _outer(i, carry):
        for slot in range(2):
            g = i * 2 + slot
            _wait(slot)
            _compute(slot)
            @pl.when(g + 2 < NBLK)
            def _():
                _start(g + 2, slot)
        return carry

    lax.fori_loop(0, NBLK // 2, _outer, 0)

    for h, r0 in ((hc, 0), (ha, NWORK), (hs, 2 * NWORK)):
        acc_row = zero
        for b in range(NBINS):
            s = jnp.sum(h[b])
            acc_row = jnp.where(lane == b, s, acc_row)
        row[...] = acc_row
        pltpu.sync_copy(row, out_hbm.at[r0 + wid])


def _finalize_body(p_ref, o_ref):
    x = p_ref[...]
    cnt = jnp.sum(x[0:NWORK, :], axis=0, keepdims=True)
    acc = jnp.sum(x[NWORK:2 * NWORK, :], axis=0, keepdims=True)
    cs = jnp.sum(x[2 * NWORK:, :], axis=0, keepdims=True)
    safe = jnp.maximum(cnt, 1.0)
    contrib = (jnp.abs(cs - acc) / safe) * (cnt * (1.0 / N))
    o_ref[0, 0] = jnp.sum(jnp.where(cnt > 0.0, contrib, 0.0))


_finalize = pl.pallas_call(
    _finalize_body,
    out_shape=jax.ShapeDtypeStruct((1, 1), jnp.float32),
    out_specs=pl.BlockSpec(memory_space=pltpu.SMEM),
)


def kernel(confidences, predictions, labels):
    p = predictions.astype(jnp.int32)
    l = labels.astype(jnp.int32)
    parts = _ece_partials(confidences, p, l)
    return _finalize(parts)[0, 0]
